# baseline (speedup 1.0000x reference)
import functools

import jax
import jax.numpy as jnp
from jax import lax
from jax.experimental import pallas as pl
from jax.experimental.pallas import tpu as pltpu
from jax.experimental.pallas import tpu_sc as plsc

_NC = 2
_NS = 16
_NW = _NC * _NS
_CHUNK = 80
_NBUF = 4
_L = 16


@functools.partial(jax.jit, static_argnames=("vocab", "dim"))
def _gather_rows(idx, weight, vocab, dim):
    n = idx.shape[0]
    per_w = n // _NW
    cpw = per_w // _CHUNK
    blocks = _CHUNK // _L
    mesh = plsc.VectorSubcoreMesh(core_axis_name="c", subcore_axis_name="s")

    @functools.partial(
        pl.kernel,
        mesh=mesh,
        compiler_params=pltpu.CompilerParams(
            use_tc_tiling_on_sc=False, needs_layout_passes=False),
        out_type=jax.ShapeDtypeStruct((n * dim,), jnp.float32),
        scratch_types=[
            pltpu.VMEM((per_w,), jnp.int32),
            pltpu.VMEM((vocab * dim,), jnp.float32),
            *[pltpu.VMEM((_CHUNK * dim,), jnp.float32) for _ in range(_NBUF)],
            *[pltpu.SemaphoreType.DMA for _ in range(2 + _NBUF)],
        ],
    )
    def k(idx_hbm, table_hbm, out_hbm, idx_v, table_v, *bufs_sems):
        bufs = bufs_sems[:_NBUF]
        sem_t, sem_i = bufs_sems[_NBUF], bufs_sems[_NBUF + 1]
        osem = bufs_sems[_NBUF + 2:]
        wid = lax.axis_index("s") * _NC + lax.axis_index("c")
        base = wid * per_w
        ct = pltpu.async_copy(table_hbm, table_v, sem_t)
        ci = pltpu.async_copy(idx_hbm.at[pl.ds(base, per_w)], idx_v, sem_i)
        ct.wait()
        ci.wait()

        o = [None] * _NBUF
        for ch in range(cpw):
            r = ch % _NBUF
            if o[r] is not None:
                o[r].wait()
            buf = bufs[r]

            def blk(b, _, ch=ch, buf=buf):
                idxv = idx_v[pl.ds(ch * _CHUNK + b * _L, _L)]
                src = idxv * dim
                dst = (b * _L + lax.iota(jnp.int32, _L)) * dim
                ones = jnp.full((_L,), 1, jnp.int32)
                for c in range(dim):
                    v = plsc.load_gather(table_v, [src])
                    plsc.store_scatter(buf, [dst], v)
                    if c + 1 < dim:
                        src = src + ones
                        dst = dst + ones
                return 0

            lax.fori_loop(0, blocks, blk, 0)
            o[r] = pltpu.async_copy(
                buf,
                out_hbm.at[
                    pl.ds((base + ch * _CHUNK) * dim, _CHUNK * dim)],
                osem[r])
        for r in range(_NBUF):
            if o[r] is not None:
                o[r].wait()

    return k(idx, weight.reshape(-1))


def kernel(tensor, weight):
    b, h = tensor.shape
    vocab, dim = weight.shape
    idx = tensor.reshape(-1).astype(jnp.int32)
    out = _gather_rows(idx, weight, vocab, dim)
    return out.reshape(b, h, dim)


# speedup vs baseline: 2.5243x; 2.5243x over previous
import functools

import jax
import jax.numpy as jnp
from jax import lax
from jax.experimental import pallas as pl
from jax.experimental.pallas import tpu as pltpu
from jax.experimental.pallas import tpu_sc as plsc

_NC = 2
_NS = 16
_NW = _NC * _NS
_CHUNK = 80
_OSEM = 4


@functools.partial(jax.jit, static_argnames=("dim",))
def _gather_rows(idx, weight, dim):
    n = idx.shape[0]
    per_w = n // _NW
    cpw = per_w // _CHUNK
    mesh = plsc.VectorSubcoreMesh(core_axis_name="c", subcore_axis_name="s")

    @functools.partial(
        pl.kernel,
        mesh=mesh,
        compiler_params=pltpu.CompilerParams(use_tc_tiling_on_sc=False),
        out_type=jax.ShapeDtypeStruct((n, dim), jnp.float32),
        scratch_types=[
            pltpu.VMEM((per_w,), jnp.int32),
            pltpu.VMEM((per_w, dim), jnp.float32),
            *[pltpu.SemaphoreType.DMA for _ in range(1 + cpw + _OSEM)],
        ],
    )
    def k(idx_hbm, table_hbm, out_hbm, idx_v, rows_v, *sems):
        isem = sems[0]
        gsem = sems[1:1 + cpw]
        osem = sems[1 + cpw:]
        wid = lax.axis_index("s") * _NC + lax.axis_index("c")
        base = wid * per_w
        pltpu.async_copy(idx_hbm.at[pl.ds(base, per_w)], idx_v, isem).wait()
        g = []
        for j in range(cpw):
            g.append(pltpu.async_copy(
                table_hbm.at[idx_v.at[pl.ds(j * _CHUNK, _CHUNK)]],
                rows_v.at[pl.ds(j * _CHUNK, _CHUNK)],
                gsem[j]))
        o = [None] * _OSEM
        for j in range(cpw):
            r = j % _OSEM
            g[j].wait()
            if o[r] is not None:
                o[r].wait()
            o[r] = pltpu.async_copy(
                rows_v.at[pl.ds(j * _CHUNK, _CHUNK)],
                out_hbm.at[pl.ds(base + j * _CHUNK, _CHUNK)],
                osem[r])
        for r in range(_OSEM):
            if o[r] is not None:
                o[r].wait()

    return k(idx, weight)


def kernel(tensor, weight):
    b, h = tensor.shape
    dim = weight.shape[1]
    idx = tensor.reshape(-1).astype(jnp.int32)
    out = _gather_rows(idx, weight, dim)
    return out.reshape(b, h, dim)


# speedup vs baseline: 2.5565x; 1.0127x over previous
import functools

import jax
import jax.numpy as jnp
from jax import lax
from jax.experimental import pallas as pl
from jax.experimental.pallas import tpu as pltpu
from jax.experimental.pallas import tpu_sc as plsc

_NC = 2
_NS = 16
_NW = _NC * _NS


@jax.jit
def _gather_rows(tensor, weight):
    nb, hist = tensor.shape
    dim = weight.shape[1]
    bpw = nb // _NW
    mesh = plsc.VectorSubcoreMesh(core_axis_name="c", subcore_axis_name="s")

    @functools.partial(
        pl.kernel,
        mesh=mesh,
        compiler_params=pltpu.CompilerParams(use_tc_tiling_on_sc=False),
        out_type=jax.ShapeDtypeStruct((nb, hist, dim), jnp.float32),
        scratch_types=[
            pltpu.VMEM((bpw, hist), jnp.int32),
            pltpu.VMEM((bpw, hist, dim), jnp.float32),
            *[pltpu.SemaphoreType.DMA for _ in range(2 + 16)],
        ],
    )
    def k(idx_hbm, table_hbm, out_hbm, idx_v, rows_v, *sems):
        isem = sems[0]
        osem = sems[1]
        gsem = sems[2:]
        wid = lax.axis_index("s") * _NC + lax.axis_index("c")
        base = wid * bpw
        pltpu.async_copy(idx_hbm.at[pl.ds(base, bpw)], idx_v, isem).wait()
        g = [None] * bpw
        for b in range(bpw):
            if b >= 16:
                g[b - 16].wait()
            g[b] = pltpu.async_copy(
                table_hbm.at[idx_v.at[b]], rows_v.at[b], gsem[b % 16])
        for b in range(max(0, bpw - 16), bpw):
            g[b].wait()
        pltpu.async_copy(rows_v, out_hbm.at[pl.ds(base, bpw)], osem).wait()

    return k(tensor, weight)


def kernel(tensor, weight):
    return _gather_rows(tensor.astype(jnp.int32), weight)
